# baseline (speedup 1.0000x reference)
import dataclasses
import functools

import jax
import jax.numpy as jnp
from jax import lax
from jax.experimental import pallas as pl
from jax.experimental.pallas import tpu as pltpu
from jax.experimental.pallas import tpu_sc as plsc

CUT_S = 5.0
CUT_L = 10.0

NC = 2
NS = 16
LN = 16
NW = NC * NS

HIGH = lax.Precision.HIGHEST


def _round_up(x, m):
    return (x + m - 1) // m * m


def _sc_params():
    cp = pltpu.CompilerParams()
    if "needs_layout_passes" in pltpu.CompilerParams.__dataclass_fields__:
        cp = dataclasses.replace(cp, needs_layout_passes=False)
    return cp


def _embed_body(z_ref, emb_ref, h_ref):
    z = z_ref[...]
    n, c = z.shape[0], emb_ref.shape[0]
    iot = lax.broadcasted_iota(jnp.int32, (n, c), 1)
    oh = (z == iot).astype(jnp.float32)
    h_ref[...] = jnp.dot(oh, emb_ref[...], precision=HIGH)


def _embed(z2, emb_table):
    n = z2.shape[0]
    h_dim = emb_table.shape[1]
    return pl.pallas_call(
        _embed_body,
        out_shape=jax.ShapeDtypeStruct((n, h_dim), jnp.float32),
    )(z2, emb_table)


def _gp_body(lab_ref, z_ref, pos_ref, num_ref, *, g, nb):
    i = pl.program_id(0)
    lab = lab_ref[...].reshape(1, nb)
    zf = z_ref[...].reshape(nb, 1).astype(jnp.float32)
    giota = lax.broadcasted_iota(jnp.int32, (g, nb), 0)
    oht = (giota == lab).astype(jnp.float32)
    pw = pos_ref[...].reshape(nb, 4) * zf
    upd = jnp.dot(oht, pw, precision=HIGH)

    @pl.when(i == 0)
    def _():
        num_ref[...] = upd

    @pl.when(i > 0)
    def _():
        num_ref[...] += upd


def _group_moments(labels, z, pos4, g):
    n = labels.shape[0]
    nb = 1000
    nblk = n // nb
    lab3 = labels.astype(jnp.int32).reshape(nblk, 1, nb)
    z3 = z.astype(jnp.int32).reshape(nblk, 1, nb)
    pos3 = pos4.reshape(nblk, nb, 4)
    return pl.pallas_call(
        functools.partial(_gp_body, g=g, nb=nb),
        grid=(nblk,),
        in_specs=[
            pl.BlockSpec((1, 1, nb), lambda i: (i, 0, 0)),
            pl.BlockSpec((1, 1, nb), lambda i: (i, 0, 0)),
            pl.BlockSpec((1, nb, 4), lambda i: (i, 0, 0)),
        ],
        out_specs=pl.BlockSpec((g, 4), lambda i: (0, 0)),
        out_shape=jax.ShapeDtypeStruct((g, 4), jnp.float32),
    )(lab3, z3, pos3)


def _pick_chunk(total, cap):
    d = min(cap, total) // 8 * 8
    while d >= 8:
        if total % d == 0:
            return d
        d -= 8
    raise ValueError(total)


def _dist_body(posx, posy, posz, numT, rc0, rc1, it0, it1, d2s, d2b,
               px, py, pz, nt, gx, gy, gz, ri, ci, db, ni, gi, db2,
               *, gp, epw, e2pw, ech):
    wid = lax.axis_index("c") * NS + lax.axis_index("s")
    pltpu.sync_copy(posx, px)
    pltpu.sync_copy(posy, py)
    pltpu.sync_copy(posz, pz)
    pltpu.sync_copy(numT, nt)

    @pl.loop(0, gp, step=LN)
    def _(k):
        den = jnp.maximum(nt[3, pl.ds(k, LN)], 1e-9)
        gx[pl.ds(k, LN)] = nt[0, pl.ds(k, LN)] / den
        gy[pl.ds(k, LN)] = nt[1, pl.ds(k, LN)] / den
        gz[pl.ds(k, LN)] = nt[2, pl.ds(k, LN)] / den

    base = wid * epw

    @pl.loop(0, epw, step=ech)
    def _(c0):
        pltpu.sync_copy(rc0.at[pl.ds(base + c0, ech)], ri)
        pltpu.sync_copy(rc1.at[pl.ds(base + c0, ech)], ci)

        @pl.loop(0, ech, step=LN)
        def _(k):
            r16 = ri[pl.ds(k, LN)]
            c16 = ci[pl.ds(k, LN)]
            dx = plsc.load_gather(px, [r16]) - plsc.load_gather(px, [c16])
            dy = plsc.load_gather(py, [r16]) - plsc.load_gather(py, [c16])
            dz = plsc.load_gather(pz, [r16]) - plsc.load_gather(pz, [c16])
            db[pl.ds(k, LN)] = dx * dx + dy * dy + dz * dz

        pltpu.sync_copy(db, d2s.at[pl.ds(base + c0, ech)])

    base2 = wid * e2pw
    pltpu.sync_copy(it0.at[pl.ds(base2, e2pw)], ni)
    pltpu.sync_copy(it1.at[pl.ds(base2, e2pw)], gi)

    @pl.loop(0, e2pw, step=LN)
    def _(k):
        n16 = ni[pl.ds(k, LN)]
        g16 = gi[pl.ds(k, LN)]
        dx = plsc.load_gather(px, [n16]) - plsc.load_gather(gx, [g16])
        dy = plsc.load_gather(py, [n16]) - plsc.load_gather(gy, [g16])
        dz = plsc.load_gather(pz, [n16]) - plsc.load_gather(gz, [g16])
        db2[pl.ds(k, LN)] = dx * dx + dy * dy + dz * dz

    pltpu.sync_copy(db2, d2b.at[pl.ds(base2, e2pw)])


def _distances(posx, posy, posz, numT, rc0, rc1, it0, it1,
               np_, gp, e_pad, e2_pad):
    epw = e_pad // NW
    e2pw = e2_pad // NW
    ech = _pick_chunk(epw, 1440)
    mesh = plsc.VectorSubcoreMesh(
        core_axis_name="c", subcore_axis_name="s",
        num_cores=NC, num_subcores=NS)
    kern = pl.kernel(
        functools.partial(_dist_body, gp=gp, epw=epw, e2pw=e2pw, ech=ech),
        out_type=(jax.ShapeDtypeStruct((e_pad,), jnp.float32),
                  jax.ShapeDtypeStruct((e2_pad,), jnp.float32)),
        mesh=mesh,
        compiler_params=_sc_params(),
        scratch_types=[
            pltpu.VMEM((np_,), jnp.float32),
            pltpu.VMEM((np_,), jnp.float32),
            pltpu.VMEM((np_,), jnp.float32),
            pltpu.VMEM((4, gp), jnp.float32),
            pltpu.VMEM((gp,), jnp.float32),
            pltpu.VMEM((gp,), jnp.float32),
            pltpu.VMEM((gp,), jnp.float32),
            pltpu.VMEM((ech,), jnp.int32),
            pltpu.VMEM((ech,), jnp.int32),
            pltpu.VMEM((ech,), jnp.float32),
            pltpu.VMEM((e2pw,), jnp.int32),
            pltpu.VMEM((e2pw,), jnp.int32),
            pltpu.VMEM((e2pw,), jnp.float32),
        ])
    return kern(posx, posy, posz, numT, rc0, rc1, it0, it1)


def _attr_body(d2_ref, means_ref, betas_ref, w_ref, b_ref, out_ref,
               *, cutoff, n_valid, be):
    i = pl.program_id(0)
    d2 = d2_ref[...].reshape(1, be)
    w = jnp.sqrt(d2 + 1e-12)
    cut = 0.5 * (jnp.cos(w * (jnp.pi / cutoff)) + 1.0)
    cut = jnp.where(w < cutoff, cut, 0.0)
    alpha = 5.0 / cutoff
    log2e = 1.4426950408889634
    u = jnp.exp2((-alpha * log2e) * w)
    means_t = means_ref[...].reshape(-1, 1)
    betas_t = betas_ref[...].reshape(-1, 1)
    t = u - means_t
    rbf_t = cut * jnp.exp2((-log2e) * betas_t * (t * t))
    attr = lax.dot_general(rbf_t, w_ref[...], (((0,), (0,)), ((), ())),
                           precision=HIGH) + b_ref[...].reshape(1, -1)
    nv_blk = n_valid // be

    @pl.when(i < nv_blk)
    def _():
        out_ref[...] = attr

    @pl.when(i >= nv_blk)
    def _():
        eidx = i * be + lax.broadcasted_iota(jnp.int32, (be, 1), 0)
        out_ref[...] = jnp.where(eidx < n_valid, attr, 0.0)


def _edge_attr(d2, means, betas, w_rbf, b_rbf, cutoff, n_valid):
    e_pad = d2.shape[0]
    h_dim = w_rbf.shape[1]
    be = 1024
    nblk = e_pad // be
    d3 = d2.reshape(nblk, 1, be)
    return pl.pallas_call(
        functools.partial(_attr_body, cutoff=cutoff, n_valid=n_valid, be=be),
        grid=(nblk,),
        in_specs=[
            pl.BlockSpec((1, 1, be), lambda i: (i, 0, 0)),
            pl.BlockSpec(means.shape, lambda i: (0,)),
            pl.BlockSpec(betas.shape, lambda i: (0,)),
            pl.BlockSpec(w_rbf.shape, lambda i: (0, 0)),
            pl.BlockSpec(b_rbf.shape, lambda i: (0,)),
        ],
        out_specs=pl.BlockSpec((be, h_dim), lambda i: (i, 0)),
        out_shape=jax.ShapeDtypeStruct((e_pad, h_dim), jnp.float32),
    )(d3, means, betas, w_rbf, b_rbf)


CHUNK = 80
ZR = 40


def _acc_pad_rows(acc_rows):
    return _round_up(acc_rows, NS * 8)


def _zero_chunks(rows_per):
    offs, o = [], 0
    while o < rows_per:
        sz = min(ZR, rows_per - o)
        offs.append((o, sz))
        o += sz
    return offs


def _agg_body(attr, tab, sct, gat, out, acc,
              isb, igb, isc, ab0, hb0, ab1, hb1, zb,
              si, sa0, sg0, sa1, sg1, ss0, ss1,
              *, acc_pad, nch):
    cid = lax.axis_index("c")
    sid = lax.axis_index("s")
    wid = cid * NS + sid

    @pl.loop(0, ZR)
    def _(i):
        for j in range(8):
            zb[i, pl.ds(j * LN, LN)] = jnp.zeros((LN,), jnp.float32)

    rows_per = acc_pad // NS
    for (zo, zs) in _zero_chunks(rows_per):
        pltpu.sync_copy(zb.at[pl.ds(0, zs), :],
                        acc.at[pl.ds(sid * rows_per + zo, zs), :])

    plsc.subcore_barrier()

    ptot = nch // NS
    pt0 = max(2, int(round(ptot * 0.38 / 2)) * 2)
    pt1 = ptot - pt0
    per_tile = jnp.where(cid == 0, pt0, pt1)
    npairs = per_tile // 2
    tile_base = jnp.where(cid == 0, sid * pt0,
                          NS * pt0 + sid * pt1) * CHUNK
    data = ((ab0, hb0, sa0, sg0, ss0), (ab1, hb1, sa1, sg1, ss1))

    def idx_copies(p, rbase):
        b0 = tile_base + (2 * p) * CHUNK
        return [(sct.at[pl.ds(b0, CHUNK)], isb.at[rbase]),
                (gat.at[pl.ds(b0, CHUNK)], igb.at[rbase]),
                (sct.at[pl.ds(b0 + CHUNK, CHUNK)], isb.at[rbase + 1]),
                (gat.at[pl.ds(b0 + CHUNK, CHUNK)], igb.at[rbase + 1])]

    def issue_in(k, slot, row):
        ab, hb, sa, sg, _ = data[slot]
        pltpu.async_copy(attr.at[pl.ds(k * CHUNK, CHUNK), :], ab, sa)
        pltpu.async_copy(tab.at[igb.at[row]], hb, sg)

    def mul_and_scatter(slot, row):
        ab, hb, sa, sg, ss = data[slot]
        pltpu.make_async_copy(attr.at[pl.ds(0, CHUNK), :], ab, sa).wait()
        pltpu.make_async_copy(tab.at[igb.at[row]], hb, sg).wait()
        for v in range(CHUNK // LN):
            isc[slot, pl.ds(v * LN, LN)] = isb[row, pl.ds(v * LN, LN)]

        @pl.loop(0, CHUNK)
        def _(i):
            for j in range(8):
                sl = (i, pl.ds(j * LN, LN))
                ab[sl] = ab[sl] * hb[sl]

        pltpu.async_copy(ab, acc.at[isc.at[slot]], ss, add=True)

    for (src, dst) in idx_copies(0, 0):
        pltpu.sync_copy(src, dst)
    issue_in(tile_base // CHUNK + 0, 0, 0)

    @pl.loop(0, npairs)
    def _(j):
        k = tile_base // CHUNK + 2 * j

        @pl.when(j > 0)
        def _():
            pltpu.make_async_copy(ab1, acc.at[isc.at[1]], ss1).wait()

        issue_in(k + 1, 1, 1)
        mul_and_scatter(0, 0)

        @pl.when(j + 1 < npairs)
        def _():
            for (src, dst) in idx_copies(j + 1, 2):
                pltpu.async_copy(src, dst, si)

        pltpu.make_async_copy(ab0, acc.at[isc.at[0]], ss0).wait()

        @pl.when(j + 1 < npairs)
        def _():
            for (src, dst) in idx_copies(j + 1, 2):
                pltpu.make_async_copy(src, dst, si).wait()
            issue_in(k + 2, 0, 2)

        mul_and_scatter(1, 1)

        @pl.when(j + 1 < npairs)
        def _():
            for r in range(2):
                for v in range(CHUNK // LN):
                    sl = pl.ds(v * LN, LN)
                    isb[r, sl] = isb[r + 2, sl]
                    igb[r, sl] = igb[r + 2, sl]

    pltpu.make_async_copy(ab1, acc.at[isc.at[1]], ss1).wait()

    plsc.subcore_barrier()
    pltpu.sync_copy(acc.at[pl.ds(sid * rows_per, rows_per), :],
                    out.at[pl.ds(cid * acc_pad + sid * rows_per,
                                 rows_per), :])


def _gather_mod_scatter(attr, tab, sct, gat, acc_rows):
    e_pad, h_dim = attr.shape
    acc_pad = _acc_pad_rows(acc_rows)
    nch = e_pad // CHUNK
    mesh = plsc.VectorSubcoreMesh(
        core_axis_name="c", subcore_axis_name="s",
        num_cores=NC, num_subcores=NS)
    kern = pl.kernel(
        functools.partial(_agg_body, acc_pad=acc_pad, nch=nch),
        out_type=jax.ShapeDtypeStruct((NC * acc_pad, h_dim), jnp.float32),
        mesh=mesh,
        scratch_types=[
            pltpu.VMEM_SHARED((acc_pad, h_dim), jnp.float32),
            pltpu.VMEM((4, CHUNK), jnp.int32),
            pltpu.VMEM((4, CHUNK), jnp.int32),
            pltpu.VMEM((2, CHUNK), jnp.int32),
            pltpu.VMEM((CHUNK, h_dim), jnp.float32),
            pltpu.VMEM((CHUNK, h_dim), jnp.float32),
            pltpu.VMEM((CHUNK, h_dim), jnp.float32),
            pltpu.VMEM((CHUNK, h_dim), jnp.float32),
            pltpu.VMEM((ZR, h_dim), jnp.float32),
            pltpu.SemaphoreType.DMA,
            pltpu.SemaphoreType.DMA,
            pltpu.SemaphoreType.DMA,
            pltpu.SemaphoreType.DMA,
            pltpu.SemaphoreType.DMA,
            pltpu.SemaphoreType.DMA,
            pltpu.SemaphoreType.DMA,
        ])
    return kern(attr, tab, sct, gat)


def _addn_body(*refs):
    o_ref = refs[-1]
    acc = refs[0][...]
    for r in refs[1:-1]:
        acc = acc + r[...]
    o_ref[...] = acc


def _add_partials(a, *parts):
    n, h_dim = a.shape
    nb = 1000
    nblk = n // nb
    return pl.pallas_call(
        _addn_body,
        grid=(nblk,),
        in_specs=[pl.BlockSpec((nb, h_dim), lambda i: (i, 0))
                  for _ in range(1 + len(parts))],
        out_specs=pl.BlockSpec((nb, h_dim), lambda i: (i, 0)),
        out_shape=jax.ShapeDtypeStruct((n, h_dim), jnp.float32),
    )(a, *parts)


def _add2_body(p0_ref, p1_ref, o_ref):
    o_ref[...] = p0_ref[...] + p1_ref[...]


def _add_partials_small(p0, p1):
    gp, h_dim = p0.shape
    return pl.pallas_call(
        _add2_body,
        grid=(1,),
        in_specs=[
            pl.BlockSpec((gp, h_dim), lambda i: (0, 0)),
            pl.BlockSpec((gp, h_dim), lambda i: (0, 0)),
        ],
        out_specs=pl.BlockSpec((gp, h_dim), lambda i: (0, 0)),
        out_shape=jax.ShapeDtypeStruct((gp, h_dim), jnp.float32),
    )(p0, p1)


def _head_body(h2_ref, l0_ref, l1_ref, w_ref, b_ref, o_ref):
    h_dim = h2_ref.shape[1]
    lf = l0_ref[...] + l1_ref[...]
    o_ref[...] = (jnp.dot(h2_ref[...], w_ref[0:h_dim, :], precision=HIGH)
                  + jnp.dot(lf, w_ref[h_dim:, :], precision=HIGH)
                  + b_ref[...].reshape(1, -1))


def _head(h2, l0, l1, w_out, b_out):
    n, h_dim = h2.shape
    nb = 1000
    nblk = n // nb
    return pl.pallas_call(
        _head_body,
        grid=(nblk,),
        in_specs=[
            pl.BlockSpec((nb, h_dim), lambda i: (i, 0)),
            pl.BlockSpec((nb, h_dim), lambda i: (i, 0)),
            pl.BlockSpec((nb, h_dim), lambda i: (i, 0)),
            pl.BlockSpec(w_out.shape, lambda i: (0, 0)),
            pl.BlockSpec(b_out.shape, lambda i: (0,)),
        ],
        out_specs=pl.BlockSpec((nb, h_dim), lambda i: (i, 0)),
        out_shape=jax.ShapeDtypeStruct((n, h_dim), jnp.float32),
    )(h2, l0, l1, w_out, b_out)


def kernel(pos, z, edge_index, labels, inter_node, inter_group, emb_table,
           means_s, betas_s, W_rbf_s, b_rbf_s,
           means_l, betas_l, W_rbf_l, b_rbf_l,
           W_out, b_out):
    n = pos.shape[0]
    e = edge_index.shape[1]
    e2 = inter_node.shape[0]
    g = 1000
    gp = _round_up(g, 64)

    nslab = 1
    e_pad = _round_up(e, NW * CHUNK * 2 * nslab)
    e2_pad = _round_up(e2, NW * CHUNK * 2)

    zi = z.astype(jnp.int32)
    z2 = zi.reshape(n, 1)
    pos4 = jnp.concatenate([pos, jnp.ones((n, 1), jnp.float32)], axis=1)
    posx, posy, posz = pos[:, 0], pos[:, 1], pos[:, 2]

    rc0 = jnp.pad(edge_index[0].astype(jnp.int32), (0, e_pad - e))
    rc1 = jnp.pad(edge_index[1].astype(jnp.int32), (0, e_pad - e))
    it0 = jnp.pad(inter_node.astype(jnp.int32), (0, e2_pad - e2))
    it1 = jnp.pad(inter_group.astype(jnp.int32), (0, e2_pad - e2))

    h = _embed(z2, emb_table)
    num = _group_moments(labels, zi, pos4, g)
    numT = jnp.pad(num.T, ((0, 0), (0, gp - g)))

    d2s, d2b = _distances(posx, posy, posz, numT, rc0, rc1, it0, it1,
                          n, gp, e_pad, e2_pad)

    attr_b = _edge_attr(d2b, means_l, betas_l, W_rbf_l, b_rbf_l, CUT_L, e2)

    ap_n = _acc_pad_rows(n)
    ns_ = e_pad // nslab
    parts = []
    for si in range(nslab):
        d2slab = lax.slice(d2s, (si * ns_,), ((si + 1) * ns_,))
        nv = max(0, min(e - si * ns_, ns_))
        attr = _edge_attr(d2slab, means_s, betas_s, W_rbf_s, b_rbf_s,
                          CUT_S, nv)
        hp = _gather_mod_scatter(attr, h,
                                 lax.slice(rc0, (si * ns_,), ((si + 1) * ns_,)),
                                 lax.slice(rc1, (si * ns_,), ((si + 1) * ns_,)),
                                 acc_rows=n)
        parts.extend([hp[:n], hp[ap_n:ap_n + n]])
    h2 = _add_partials(h, *parts)

    ap_g = _acc_pad_rows(gp)
    gparts = _gather_mod_scatter(attr_b, h2, it1, it0, acc_rows=gp)
    g_agg = _add_partials_small(gparts[:ap_g], gparts[ap_g:])

    lparts = _gather_mod_scatter(attr_b, g_agg, it0, it1, acc_rows=n)

    return _head(h2, lparts[:n], lparts[ap_n:ap_n + n], W_out, b_out)


# speedup vs baseline: 1.0401x; 1.0401x over previous
import dataclasses
import functools

import jax
import jax.numpy as jnp
from jax import lax
from jax.experimental import pallas as pl
from jax.experimental.pallas import tpu as pltpu
from jax.experimental.pallas import tpu_sc as plsc

CUT_S = 5.0
CUT_L = 10.0

NC = 2
NS = 16
LN = 16
NW = NC * NS

HIGH = lax.Precision.HIGHEST


def _round_up(x, m):
    return (x + m - 1) // m * m


def _sc_params():
    cp = pltpu.CompilerParams()
    if "needs_layout_passes" in pltpu.CompilerParams.__dataclass_fields__:
        cp = dataclasses.replace(cp, needs_layout_passes=False)
    return cp


def _embed_body(z_ref, emb_ref, h_ref):
    z = z_ref[...]
    n, c = z.shape[0], emb_ref.shape[0]
    iot = lax.broadcasted_iota(jnp.int32, (n, c), 1)
    oh = (z == iot).astype(jnp.float32)
    h_ref[...] = jnp.dot(oh, emb_ref[...], precision=HIGH)


def _embed(z2, emb_table):
    n = z2.shape[0]
    h_dim = emb_table.shape[1]
    return pl.pallas_call(
        _embed_body,
        out_shape=jax.ShapeDtypeStruct((n, h_dim), jnp.float32),
    )(z2, emb_table)


def _gp_body(lab_ref, z_ref, pos_ref, num_ref, *, g, nb):
    i = pl.program_id(0)
    lab = lab_ref[...].reshape(1, nb)
    zf = z_ref[...].reshape(nb, 1).astype(jnp.float32)
    giota = lax.broadcasted_iota(jnp.int32, (g, nb), 0)
    oht = (giota == lab).astype(jnp.float32)
    pw = pos_ref[...].reshape(nb, 4) * zf
    upd = jnp.dot(oht, pw, precision=HIGH)

    @pl.when(i == 0)
    def _():
        num_ref[...] = upd

    @pl.when(i > 0)
    def _():
        num_ref[...] += upd


def _group_moments(labels, z, pos4, g):
    n = labels.shape[0]
    nb = 1000
    nblk = n // nb
    lab3 = labels.astype(jnp.int32).reshape(nblk, 1, nb)
    z3 = z.astype(jnp.int32).reshape(nblk, 1, nb)
    pos3 = pos4.reshape(nblk, nb, 4)
    return pl.pallas_call(
        functools.partial(_gp_body, g=g, nb=nb),
        grid=(nblk,),
        in_specs=[
            pl.BlockSpec((1, 1, nb), lambda i: (i, 0, 0)),
            pl.BlockSpec((1, 1, nb), lambda i: (i, 0, 0)),
            pl.BlockSpec((1, nb, 4), lambda i: (i, 0, 0)),
        ],
        out_specs=pl.BlockSpec((g, 4), lambda i: (0, 0)),
        out_shape=jax.ShapeDtypeStruct((g, 4), jnp.float32),
    )(lab3, z3, pos3)


def _pick_chunk(total, cap):
    d = min(cap, total) // 8 * 8
    while d >= 8:
        if total % d == 0:
            return d
        d -= 8
    raise ValueError(total)


def _dist_body(posx, posy, posz, numT, rc0, rc1, it0, it1, d2s, d2b,
               px, py, pz, nt, gx, gy, gz, ri, ci, db, ni, gi, db2,
               *, gp, epw, e2pw, ech):
    wid = lax.axis_index("c") * NS + lax.axis_index("s")
    pltpu.sync_copy(posx, px)
    pltpu.sync_copy(posy, py)
    pltpu.sync_copy(posz, pz)
    pltpu.sync_copy(numT, nt)

    @pl.loop(0, gp, step=LN)
    def _(k):
        den = jnp.maximum(nt[3, pl.ds(k, LN)], 1e-9)
        gx[pl.ds(k, LN)] = nt[0, pl.ds(k, LN)] / den
        gy[pl.ds(k, LN)] = nt[1, pl.ds(k, LN)] / den
        gz[pl.ds(k, LN)] = nt[2, pl.ds(k, LN)] / den

    base = wid * epw

    @pl.loop(0, epw, step=ech)
    def _(c0):
        pltpu.sync_copy(rc0.at[pl.ds(base + c0, ech)], ri)
        pltpu.sync_copy(rc1.at[pl.ds(base + c0, ech)], ci)

        @pl.loop(0, ech, step=LN)
        def _(k):
            r16 = ri[pl.ds(k, LN)]
            c16 = ci[pl.ds(k, LN)]
            dx = plsc.load_gather(px, [r16]) - plsc.load_gather(px, [c16])
            dy = plsc.load_gather(py, [r16]) - plsc.load_gather(py, [c16])
            dz = plsc.load_gather(pz, [r16]) - plsc.load_gather(pz, [c16])
            db[pl.ds(k, LN)] = dx * dx + dy * dy + dz * dz

        pltpu.sync_copy(db, d2s.at[pl.ds(base + c0, ech)])

    base2 = wid * e2pw
    pltpu.sync_copy(it0.at[pl.ds(base2, e2pw)], ni)
    pltpu.sync_copy(it1.at[pl.ds(base2, e2pw)], gi)

    @pl.loop(0, e2pw, step=LN)
    def _(k):
        n16 = ni[pl.ds(k, LN)]
        g16 = gi[pl.ds(k, LN)]
        dx = plsc.load_gather(px, [n16]) - plsc.load_gather(gx, [g16])
        dy = plsc.load_gather(py, [n16]) - plsc.load_gather(gy, [g16])
        dz = plsc.load_gather(pz, [n16]) - plsc.load_gather(gz, [g16])
        db2[pl.ds(k, LN)] = dx * dx + dy * dy + dz * dz

    pltpu.sync_copy(db2, d2b.at[pl.ds(base2, e2pw)])


def _distances(posx, posy, posz, numT, rc0, rc1, it0, it1,
               np_, gp, e_pad, e2_pad):
    epw = e_pad // NW
    e2pw = e2_pad // NW
    ech = _pick_chunk(epw, 1440)
    mesh = plsc.VectorSubcoreMesh(
        core_axis_name="c", subcore_axis_name="s",
        num_cores=NC, num_subcores=NS)
    kern = pl.kernel(
        functools.partial(_dist_body, gp=gp, epw=epw, e2pw=e2pw, ech=ech),
        out_type=(jax.ShapeDtypeStruct((e_pad,), jnp.float32),
                  jax.ShapeDtypeStruct((e2_pad,), jnp.float32)),
        mesh=mesh,
        compiler_params=_sc_params(),
        scratch_types=[
            pltpu.VMEM((np_,), jnp.float32),
            pltpu.VMEM((np_,), jnp.float32),
            pltpu.VMEM((np_,), jnp.float32),
            pltpu.VMEM((4, gp), jnp.float32),
            pltpu.VMEM((gp,), jnp.float32),
            pltpu.VMEM((gp,), jnp.float32),
            pltpu.VMEM((gp,), jnp.float32),
            pltpu.VMEM((ech,), jnp.int32),
            pltpu.VMEM((ech,), jnp.int32),
            pltpu.VMEM((ech,), jnp.float32),
            pltpu.VMEM((e2pw,), jnp.int32),
            pltpu.VMEM((e2pw,), jnp.int32),
            pltpu.VMEM((e2pw,), jnp.float32),
        ])
    return kern(posx, posy, posz, numT, rc0, rc1, it0, it1)


def _attr_body(d2_ref, means_ref, betas_ref, w_ref, b_ref, out_ref,
               *, cutoff, n_valid, be):
    i = pl.program_id(0)
    d2 = d2_ref[...].reshape(1, be)
    w = jnp.sqrt(d2 + 1e-12)
    cut = 0.5 * (jnp.cos(w * (jnp.pi / cutoff)) + 1.0)
    cut = jnp.where(w < cutoff, cut, 0.0)
    alpha = 5.0 / cutoff
    log2e = 1.4426950408889634
    u = jnp.exp2((-alpha * log2e) * w)
    means_t = means_ref[...].reshape(-1, 1)
    betas_t = betas_ref[...].reshape(-1, 1)
    t = u - means_t
    rbf_t = cut * jnp.exp2((-log2e) * betas_t * (t * t))
    attr = lax.dot_general(rbf_t, w_ref[...], (((0,), (0,)), ((), ())),
                           precision=HIGH) + b_ref[...].reshape(1, -1)
    nv_blk = n_valid // be

    @pl.when(i < nv_blk)
    def _():
        out_ref[...] = attr

    @pl.when(i >= nv_blk)
    def _():
        eidx = i * be + lax.broadcasted_iota(jnp.int32, (be, 1), 0)
        out_ref[...] = jnp.where(eidx < n_valid, attr, 0.0)


def _edge_attr(d2, means, betas, w_rbf, b_rbf, cutoff, n_valid):
    e_pad = d2.shape[0]
    h_dim = w_rbf.shape[1]
    be = 1024
    nblk = e_pad // be
    d3 = d2.reshape(nblk, 1, be)
    return pl.pallas_call(
        functools.partial(_attr_body, cutoff=cutoff, n_valid=n_valid, be=be),
        grid=(nblk,),
        in_specs=[
            pl.BlockSpec((1, 1, be), lambda i: (i, 0, 0)),
            pl.BlockSpec(means.shape, lambda i: (0,)),
            pl.BlockSpec(betas.shape, lambda i: (0,)),
            pl.BlockSpec(w_rbf.shape, lambda i: (0, 0)),
            pl.BlockSpec(b_rbf.shape, lambda i: (0,)),
        ],
        out_specs=pl.BlockSpec((be, h_dim), lambda i: (i, 0)),
        out_shape=jax.ShapeDtypeStruct((e_pad, h_dim), jnp.float32),
    )(d3, means, betas, w_rbf, b_rbf)


CHUNK = 80
ZR = 40


def _acc_pad_rows(acc_rows):
    return _round_up(acc_rows, NS * 8)


def _zero_chunks(rows_per):
    offs, o = [], 0
    while o < rows_per:
        sz = min(ZR, rows_per - o)
        offs.append((o, sz))
        o += sz
    return offs


def _agg_body(attr, tab, sct, gat, out, acc,
              isb, igb, isc, ab0, hb0, ab1, hb1, zb,
              si, sa0, sg0, sa1, sg1, ss0, ss1,
              *, acc_pad, nch):
    cid = lax.axis_index("c")
    sid = lax.axis_index("s")
    wid = cid * NS + sid

    @pl.loop(0, ZR)
    def _(i):
        for j in range(8):
            zb[i, pl.ds(j * LN, LN)] = jnp.zeros((LN,), jnp.float32)

    rows_per = acc_pad // NS
    for (zo, zs) in _zero_chunks(rows_per):
        pltpu.sync_copy(zb.at[pl.ds(0, zs), :],
                        acc.at[pl.ds(sid * rows_per + zo, zs), :])

    plsc.subcore_barrier()

    ptot = nch // NS
    pt0 = ptot // 2
    pt1 = ptot - pt0
    per_tile = jnp.where(cid == 0, pt0, pt1)
    npairs = per_tile // 2
    tile_base = jnp.where(cid == 0, sid * pt0,
                          NS * pt0 + sid * pt1) * CHUNK
    data = ((ab0, hb0, sa0, sg0, ss0), (ab1, hb1, sa1, sg1, ss1))

    def idx_copies(p, rbase):
        b0 = tile_base + (2 * p) * CHUNK
        return [(sct.at[pl.ds(b0, CHUNK)], isb.at[rbase]),
                (gat.at[pl.ds(b0, CHUNK)], igb.at[rbase]),
                (sct.at[pl.ds(b0 + CHUNK, CHUNK)], isb.at[rbase + 1]),
                (gat.at[pl.ds(b0 + CHUNK, CHUNK)], igb.at[rbase + 1])]

    def issue_in(k, slot, row):
        ab, hb, sa, sg, _ = data[slot]
        pltpu.async_copy(attr.at[pl.ds(k * CHUNK, CHUNK), :], ab, sa)
        pltpu.async_copy(tab.at[igb.at[row]], hb, sg)

    def mul_and_scatter(slot, row):
        ab, hb, sa, sg, ss = data[slot]
        pltpu.make_async_copy(attr.at[pl.ds(0, CHUNK), :], ab, sa).wait()
        pltpu.make_async_copy(tab.at[igb.at[row]], hb, sg).wait()
        for v in range(CHUNK // LN):
            isc[slot, pl.ds(v * LN, LN)] = isb[row, pl.ds(v * LN, LN)]

        @pl.loop(0, CHUNK)
        def _(i):
            for j in range(8):
                sl = (i, pl.ds(j * LN, LN))
                ab[sl] = ab[sl] * hb[sl]

        pltpu.async_copy(ab, acc.at[isc.at[slot]], ss, add=True)

    for (src, dst) in idx_copies(0, 0):
        pltpu.sync_copy(src, dst)
    issue_in(tile_base // CHUNK + 0, 0, 0)

    @pl.loop(0, npairs)
    def _(j):
        k = tile_base // CHUNK + 2 * j

        @pl.when(j > 0)
        def _():
            pltpu.make_async_copy(ab1, acc.at[isc.at[1]], ss1).wait()

        issue_in(k + 1, 1, 1)
        mul_and_scatter(0, 0)

        @pl.when(j + 1 < npairs)
        def _():
            for (src, dst) in idx_copies(j + 1, 2):
                pltpu.async_copy(src, dst, si)

        pltpu.make_async_copy(ab0, acc.at[isc.at[0]], ss0).wait()

        @pl.when(j + 1 < npairs)
        def _():
            for (src, dst) in idx_copies(j + 1, 2):
                pltpu.make_async_copy(src, dst, si).wait()
            issue_in(k + 2, 0, 2)

        mul_and_scatter(1, 1)

        @pl.when(j + 1 < npairs)
        def _():
            for r in range(2):
                for v in range(CHUNK // LN):
                    sl = pl.ds(v * LN, LN)
                    isb[r, sl] = isb[r + 2, sl]
                    igb[r, sl] = igb[r + 2, sl]

    pltpu.make_async_copy(ab1, acc.at[isc.at[1]], ss1).wait()

    plsc.subcore_barrier()
    pltpu.sync_copy(acc.at[pl.ds(sid * rows_per, rows_per), :],
                    out.at[pl.ds(cid * acc_pad + sid * rows_per,
                                 rows_per), :])


def _gather_mod_scatter(attr, tab, sct, gat, acc_rows):
    e_pad, h_dim = attr.shape
    acc_pad = _acc_pad_rows(acc_rows)
    nch = e_pad // CHUNK
    mesh = plsc.VectorSubcoreMesh(
        core_axis_name="c", subcore_axis_name="s",
        num_cores=NC, num_subcores=NS)
    kern = pl.kernel(
        functools.partial(_agg_body, acc_pad=acc_pad, nch=nch),
        out_type=jax.ShapeDtypeStruct((NC * acc_pad, h_dim), jnp.float32),
        mesh=mesh,
        scratch_types=[
            pltpu.VMEM_SHARED((acc_pad, h_dim), jnp.float32),
            pltpu.VMEM((4, CHUNK), jnp.int32),
            pltpu.VMEM((4, CHUNK), jnp.int32),
            pltpu.VMEM((2, CHUNK), jnp.int32),
            pltpu.VMEM((CHUNK, h_dim), jnp.float32),
            pltpu.VMEM((CHUNK, h_dim), jnp.float32),
            pltpu.VMEM((CHUNK, h_dim), jnp.float32),
            pltpu.VMEM((CHUNK, h_dim), jnp.float32),
            pltpu.VMEM((ZR, h_dim), jnp.float32),
            pltpu.SemaphoreType.DMA,
            pltpu.SemaphoreType.DMA,
            pltpu.SemaphoreType.DMA,
            pltpu.SemaphoreType.DMA,
            pltpu.SemaphoreType.DMA,
            pltpu.SemaphoreType.DMA,
            pltpu.SemaphoreType.DMA,
        ])
    return kern(attr, tab, sct, gat)


def _addn_body(*refs):
    o_ref = refs[-1]
    acc = refs[0][...]
    for r in refs[1:-1]:
        acc = acc + r[...]
    o_ref[...] = acc


def _add_partials(a, *parts):
    n, h_dim = a.shape
    nb = 1000
    nblk = n // nb
    return pl.pallas_call(
        _addn_body,
        grid=(nblk,),
        in_specs=[pl.BlockSpec((nb, h_dim), lambda i: (i, 0))
                  for _ in range(1 + len(parts))],
        out_specs=pl.BlockSpec((nb, h_dim), lambda i: (i, 0)),
        out_shape=jax.ShapeDtypeStruct((n, h_dim), jnp.float32),
    )(a, *parts)


def _add2_body(p0_ref, p1_ref, o_ref):
    o_ref[...] = p0_ref[...] + p1_ref[...]


def _add_partials_small(p0, p1):
    gp, h_dim = p0.shape
    return pl.pallas_call(
        _add2_body,
        grid=(1,),
        in_specs=[
            pl.BlockSpec((gp, h_dim), lambda i: (0, 0)),
            pl.BlockSpec((gp, h_dim), lambda i: (0, 0)),
        ],
        out_specs=pl.BlockSpec((gp, h_dim), lambda i: (0, 0)),
        out_shape=jax.ShapeDtypeStruct((gp, h_dim), jnp.float32),
    )(p0, p1)


def _head_body(h2_ref, l0_ref, l1_ref, w_ref, b_ref, o_ref):
    h_dim = h2_ref.shape[1]
    lf = l0_ref[...] + l1_ref[...]
    o_ref[...] = (jnp.dot(h2_ref[...], w_ref[0:h_dim, :], precision=HIGH)
                  + jnp.dot(lf, w_ref[h_dim:, :], precision=HIGH)
                  + b_ref[...].reshape(1, -1))


def _head(h2, l0, l1, w_out, b_out):
    n, h_dim = h2.shape
    nb = 1000
    nblk = n // nb
    return pl.pallas_call(
        _head_body,
        grid=(nblk,),
        in_specs=[
            pl.BlockSpec((nb, h_dim), lambda i: (i, 0)),
            pl.BlockSpec((nb, h_dim), lambda i: (i, 0)),
            pl.BlockSpec((nb, h_dim), lambda i: (i, 0)),
            pl.BlockSpec(w_out.shape, lambda i: (0, 0)),
            pl.BlockSpec(b_out.shape, lambda i: (0,)),
        ],
        out_specs=pl.BlockSpec((nb, h_dim), lambda i: (i, 0)),
        out_shape=jax.ShapeDtypeStruct((n, h_dim), jnp.float32),
    )(h2, l0, l1, w_out, b_out)


def kernel(pos, z, edge_index, labels, inter_node, inter_group, emb_table,
           means_s, betas_s, W_rbf_s, b_rbf_s,
           means_l, betas_l, W_rbf_l, b_rbf_l,
           W_out, b_out):
    n = pos.shape[0]
    e = edge_index.shape[1]
    e2 = inter_node.shape[0]
    g = 1000
    gp = _round_up(g, 64)

    nslab = 1
    e_pad = _round_up(e, NW * CHUNK * 2 * nslab)
    e2_pad = _round_up(e2, NW * CHUNK * 2)

    zi = z.astype(jnp.int32)
    z2 = zi.reshape(n, 1)
    pos4 = jnp.concatenate([pos, jnp.ones((n, 1), jnp.float32)], axis=1)
    posx, posy, posz = pos[:, 0], pos[:, 1], pos[:, 2]

    rc0 = jnp.pad(edge_index[0].astype(jnp.int32), (0, e_pad - e))
    rc1 = jnp.pad(edge_index[1].astype(jnp.int32), (0, e_pad - e))
    it0 = jnp.pad(inter_node.astype(jnp.int32), (0, e2_pad - e2))
    it1 = jnp.pad(inter_group.astype(jnp.int32), (0, e2_pad - e2))

    h = _embed(z2, emb_table)
    num = _group_moments(labels, zi, pos4, g)
    numT = jnp.pad(num.T, ((0, 0), (0, gp - g)))

    d2s, d2b = _distances(posx, posy, posz, numT, rc0, rc1, it0, it1,
                          n, gp, e_pad, e2_pad)

    attr_b = _edge_attr(d2b, means_l, betas_l, W_rbf_l, b_rbf_l, CUT_L, e2)

    ap_n = _acc_pad_rows(n)
    ns_ = e_pad // nslab
    parts = []
    for si in range(nslab):
        d2slab = lax.slice(d2s, (si * ns_,), ((si + 1) * ns_,))
        nv = max(0, min(e - si * ns_, ns_))
        attr = _edge_attr(d2slab, means_s, betas_s, W_rbf_s, b_rbf_s,
                          CUT_S, nv)
        hp = _gather_mod_scatter(attr, h,
                                 lax.slice(rc0, (si * ns_,), ((si + 1) * ns_,)),
                                 lax.slice(rc1, (si * ns_,), ((si + 1) * ns_,)),
                                 acc_rows=n)
        parts.extend([hp[:n], hp[ap_n:ap_n + n]])
    h2 = _add_partials(h, *parts)

    ap_g = _acc_pad_rows(gp)
    gparts = _gather_mod_scatter(attr_b, h2, it1, it0, acc_rows=gp)
    g_agg = _add_partials_small(gparts[:ap_g], gparts[ap_g:])

    lparts = _gather_mod_scatter(attr_b, g_agg, it0, it1, acc_rows=n)

    return _head(h2, lparts[:n], lparts[ap_n:ap_n + n], W_out, b_out)


# speedup vs baseline: 1.0903x; 1.0483x over previous
import dataclasses
import functools

import jax
import jax.numpy as jnp
from jax import lax
from jax.experimental import pallas as pl
from jax.experimental.pallas import tpu as pltpu
from jax.experimental.pallas import tpu_sc as plsc

CUT_S = 5.0
CUT_L = 10.0

NC = 2
NS = 16
LN = 16
NW = NC * NS

HIGH = lax.Precision.HIGHEST


def _round_up(x, m):
    return (x + m - 1) // m * m


def _sc_params():
    cp = pltpu.CompilerParams()
    if "needs_layout_passes" in pltpu.CompilerParams.__dataclass_fields__:
        cp = dataclasses.replace(cp, needs_layout_passes=False)
    return cp


def _embed_body(z_ref, emb_ref, h_ref):
    z = z_ref[...]
    n, c = z.shape[0], emb_ref.shape[0]
    iot = lax.broadcasted_iota(jnp.int32, (n, c), 1)
    oh = (z == iot).astype(jnp.float32)
    h_ref[...] = jnp.dot(oh, emb_ref[...], precision=HIGH)


def _embed(z2, emb_table):
    n = z2.shape[0]
    h_dim = emb_table.shape[1]
    return pl.pallas_call(
        _embed_body,
        out_shape=jax.ShapeDtypeStruct((n, h_dim), jnp.float32),
    )(z2, emb_table)


def _gp_body(lab_ref, z_ref, pos_ref, num_ref, *, g, nb):
    i = pl.program_id(0)
    lab = lab_ref[...].reshape(1, nb)
    zf = z_ref[...].reshape(nb, 1).astype(jnp.float32)
    giota = lax.broadcasted_iota(jnp.int32, (g, nb), 0)
    oht = (giota == lab).astype(jnp.float32)
    pw = pos_ref[...].reshape(nb, 4) * zf
    upd = jnp.dot(oht, pw, precision=HIGH)

    @pl.when(i == 0)
    def _():
        num_ref[...] = upd

    @pl.when(i > 0)
    def _():
        num_ref[...] += upd


def _group_moments(labels, z, pos4, g):
    n = labels.shape[0]
    nb = 1000
    nblk = n // nb
    lab3 = labels.astype(jnp.int32).reshape(nblk, 1, nb)
    z3 = z.astype(jnp.int32).reshape(nblk, 1, nb)
    pos3 = pos4.reshape(nblk, nb, 4)
    return pl.pallas_call(
        functools.partial(_gp_body, g=g, nb=nb),
        grid=(nblk,),
        in_specs=[
            pl.BlockSpec((1, 1, nb), lambda i: (i, 0, 0)),
            pl.BlockSpec((1, 1, nb), lambda i: (i, 0, 0)),
            pl.BlockSpec((1, nb, 4), lambda i: (i, 0, 0)),
        ],
        out_specs=pl.BlockSpec((g, 4), lambda i: (0, 0)),
        out_shape=jax.ShapeDtypeStruct((g, 4), jnp.float32),
    )(lab3, z3, pos3)


def _pick_chunk(total, cap):
    d = min(cap, total) // 8 * 8
    while d >= 8:
        if total % d == 0:
            return d
        d -= 8
    raise ValueError(total)


def _dist_body(posx, posy, posz, numT, rc0, rc1, it0, it1, d2s, d2b,
               px, py, pz, nt, gx, gy, gz, ri, ci, db, ni, gi, db2,
               *, gp, epw, e2pw, ech):
    wid = lax.axis_index("c") * NS + lax.axis_index("s")
    pltpu.sync_copy(posx, px)
    pltpu.sync_copy(posy, py)
    pltpu.sync_copy(posz, pz)
    pltpu.sync_copy(numT, nt)

    @pl.loop(0, gp, step=LN)
    def _(k):
        den = jnp.maximum(nt[3, pl.ds(k, LN)], 1e-9)
        gx[pl.ds(k, LN)] = nt[0, pl.ds(k, LN)] / den
        gy[pl.ds(k, LN)] = nt[1, pl.ds(k, LN)] / den
        gz[pl.ds(k, LN)] = nt[2, pl.ds(k, LN)] / den

    base = wid * epw

    @pl.loop(0, epw, step=ech)
    def _(c0):
        pltpu.sync_copy(rc0.at[pl.ds(base + c0, ech)], ri)
        pltpu.sync_copy(rc1.at[pl.ds(base + c0, ech)], ci)

        @pl.loop(0, ech, step=LN)
        def _(k):
            r16 = ri[pl.ds(k, LN)]
            c16 = ci[pl.ds(k, LN)]
            dx = plsc.load_gather(px, [r16]) - plsc.load_gather(px, [c16])
            dy = plsc.load_gather(py, [r16]) - plsc.load_gather(py, [c16])
            dz = plsc.load_gather(pz, [r16]) - plsc.load_gather(pz, [c16])
            db[pl.ds(k, LN)] = dx * dx + dy * dy + dz * dz

        pltpu.sync_copy(db, d2s.at[pl.ds(base + c0, ech)])

    base2 = wid * e2pw
    pltpu.sync_copy(it0.at[pl.ds(base2, e2pw)], ni)
    pltpu.sync_copy(it1.at[pl.ds(base2, e2pw)], gi)

    @pl.loop(0, e2pw, step=LN)
    def _(k):
        n16 = ni[pl.ds(k, LN)]
        g16 = gi[pl.ds(k, LN)]
        dx = plsc.load_gather(px, [n16]) - plsc.load_gather(gx, [g16])
        dy = plsc.load_gather(py, [n16]) - plsc.load_gather(gy, [g16])
        dz = plsc.load_gather(pz, [n16]) - plsc.load_gather(gz, [g16])
        db2[pl.ds(k, LN)] = dx * dx + dy * dy + dz * dz

    pltpu.sync_copy(db2, d2b.at[pl.ds(base2, e2pw)])


def _distances(posx, posy, posz, numT, rc0, rc1, it0, it1,
               np_, gp, e_pad, e2_pad):
    epw = e_pad // NW
    e2pw = e2_pad // NW
    ech = _pick_chunk(epw, 1440)
    mesh = plsc.VectorSubcoreMesh(
        core_axis_name="c", subcore_axis_name="s",
        num_cores=NC, num_subcores=NS)
    kern = pl.kernel(
        functools.partial(_dist_body, gp=gp, epw=epw, e2pw=e2pw, ech=ech),
        out_type=(jax.ShapeDtypeStruct((e_pad,), jnp.float32),
                  jax.ShapeDtypeStruct((e2_pad,), jnp.float32)),
        mesh=mesh,
        compiler_params=_sc_params(),
        scratch_types=[
            pltpu.VMEM((np_,), jnp.float32),
            pltpu.VMEM((np_,), jnp.float32),
            pltpu.VMEM((np_,), jnp.float32),
            pltpu.VMEM((4, gp), jnp.float32),
            pltpu.VMEM((gp,), jnp.float32),
            pltpu.VMEM((gp,), jnp.float32),
            pltpu.VMEM((gp,), jnp.float32),
            pltpu.VMEM((ech,), jnp.int32),
            pltpu.VMEM((ech,), jnp.int32),
            pltpu.VMEM((ech,), jnp.float32),
            pltpu.VMEM((e2pw,), jnp.int32),
            pltpu.VMEM((e2pw,), jnp.int32),
            pltpu.VMEM((e2pw,), jnp.float32),
        ])
    return kern(posx, posy, posz, numT, rc0, rc1, it0, it1)


def _attr_body(d2_ref, means_ref, betas_ref, w_ref, b_ref, out_ref,
               *, cutoff, n_valid, be):
    i = pl.program_id(0)
    d2 = d2_ref[...].reshape(1, be)
    w = jnp.sqrt(d2 + 1e-12)
    cut = 0.5 * (jnp.cos(w * (jnp.pi / cutoff)) + 1.0)
    cut = jnp.where(w < cutoff, cut, 0.0)
    alpha = 5.0 / cutoff
    log2e = 1.4426950408889634
    u = jnp.exp2((-alpha * log2e) * w)
    means_t = means_ref[...].reshape(-1, 1)
    betas_t = betas_ref[...].reshape(-1, 1)
    t = u - means_t
    rbf_t = cut * jnp.exp2((-log2e) * betas_t * (t * t))
    attr = lax.dot_general(rbf_t, w_ref[...], (((0,), (0,)), ((), ())),
                           precision=HIGH)
    attr = attr + b_ref[...].reshape(1, -1)
    nv_blk = n_valid // be

    @pl.when(i < nv_blk)
    def _():
        out_ref[...] = attr

    @pl.when(i >= nv_blk)
    def _():
        eidx = i * be + lax.broadcasted_iota(jnp.int32, (be, 1), 0)
        out_ref[...] = jnp.where(eidx < n_valid, attr, 0.0)


def _edge_attr(d2, means, betas, w_rbf, b_rbf, cutoff, n_valid):
    e_pad = d2.shape[0]
    h_dim = w_rbf.shape[1]
    be = 1536 if e_pad % 1536 == 0 else 1024
    nblk = e_pad // be
    d3 = d2.reshape(nblk, 1, be)
    return pl.pallas_call(
        functools.partial(_attr_body, cutoff=cutoff, n_valid=n_valid, be=be),
        grid=(nblk,),
        in_specs=[
            pl.BlockSpec((1, 1, be), lambda i: (i, 0, 0)),
            pl.BlockSpec(means.shape, lambda i: (0,)),
            pl.BlockSpec(betas.shape, lambda i: (0,)),
            pl.BlockSpec(w_rbf.shape, lambda i: (0, 0)),
            pl.BlockSpec(b_rbf.shape, lambda i: (0,)),
        ],
        out_specs=pl.BlockSpec((be, h_dim), lambda i: (i, 0)),
        out_shape=jax.ShapeDtypeStruct((e_pad, h_dim), jnp.float32),
    )(d3, means, betas, w_rbf, b_rbf)


CHUNK = 80
ZR = 40


def _acc_pad_rows(acc_rows):
    return _round_up(acc_rows, NS * 8)


def _zero_chunks(rows_per):
    offs, o = [], 0
    while o < rows_per:
        sz = min(ZR, rows_per - o)
        offs.append((o, sz))
        o += sz
    return offs


def _agg_body(attr, tab, sct, gat, out, acc,
              isb, igb, isc, ab0, hb0, ab1, hb1, zb,
              si, sa0, sg0, sa1, sg1, ss0, ss1,
              *, acc_pad, nch):
    cid = lax.axis_index("c")
    sid = lax.axis_index("s")
    wid = cid * NS + sid

    @pl.loop(0, ZR)
    def _(i):
        for j in range(8):
            zb[i, pl.ds(j * LN, LN)] = jnp.zeros((LN,), jnp.float32)

    rows_per = acc_pad // NS
    for (zo, zs) in _zero_chunks(rows_per):
        pltpu.sync_copy(zb.at[pl.ds(0, zs), :],
                        acc.at[pl.ds(sid * rows_per + zo, zs), :])

    plsc.subcore_barrier()

    ptot = nch // NS
    pt0 = ptot // 2
    pt1 = ptot - pt0
    per_tile = jnp.where(cid == 0, pt0, pt1)
    npairs = per_tile // 2
    tile_base = jnp.where(cid == 0, sid * pt0,
                          NS * pt0 + sid * pt1) * CHUNK
    data = ((ab0, hb0, sa0, sg0, ss0), (ab1, hb1, sa1, sg1, ss1))

    def idx_copies(p, rbase):
        b0 = tile_base + (2 * p) * CHUNK
        return [(sct.at[pl.ds(b0, CHUNK)], isb.at[rbase]),
                (gat.at[pl.ds(b0, CHUNK)], igb.at[rbase]),
                (sct.at[pl.ds(b0 + CHUNK, CHUNK)], isb.at[rbase + 1]),
                (gat.at[pl.ds(b0 + CHUNK, CHUNK)], igb.at[rbase + 1])]

    def issue_in(k, slot, row):
        ab, hb, sa, sg, _ = data[slot]
        pltpu.async_copy(attr.at[pl.ds(k * CHUNK, CHUNK), :], ab, sa)
        pltpu.async_copy(tab.at[igb.at[row]], hb, sg)

    def mul_and_scatter(slot, row):
        ab, hb, sa, sg, ss = data[slot]
        pltpu.make_async_copy(attr.at[pl.ds(0, CHUNK), :], ab, sa).wait()
        pltpu.make_async_copy(tab.at[igb.at[row]], hb, sg).wait()
        for v in range(CHUNK // LN):
            isc[slot, pl.ds(v * LN, LN)] = isb[row, pl.ds(v * LN, LN)]

        @pl.loop(0, CHUNK)
        def _(i):
            for j in range(8):
                sl = (i, pl.ds(j * LN, LN))
                ab[sl] = ab[sl] * hb[sl]

        pltpu.async_copy(ab, acc.at[isc.at[slot]], ss, add=True)

    for (src, dst) in idx_copies(0, 0):
        pltpu.sync_copy(src, dst)
    issue_in(tile_base // CHUNK + 0, 0, 0)

    @pl.loop(0, npairs)
    def _(j):
        k = tile_base // CHUNK + 2 * j

        @pl.when(j > 0)
        def _():
            pltpu.make_async_copy(ab1, acc.at[isc.at[1]], ss1).wait()

        issue_in(k + 1, 1, 1)
        mul_and_scatter(0, 0)

        @pl.when(j + 1 < npairs)
        def _():
            for (src, dst) in idx_copies(j + 1, 2):
                pltpu.async_copy(src, dst, si)

        pltpu.make_async_copy(ab0, acc.at[isc.at[0]], ss0).wait()

        @pl.when(j + 1 < npairs)
        def _():
            for (src, dst) in idx_copies(j + 1, 2):
                pltpu.make_async_copy(src, dst, si).wait()
            issue_in(k + 2, 0, 2)

        mul_and_scatter(1, 1)

        @pl.when(j + 1 < npairs)
        def _():
            for r in range(2):
                for v in range(CHUNK // LN):
                    sl = pl.ds(v * LN, LN)
                    isb[r, sl] = isb[r + 2, sl]
                    igb[r, sl] = igb[r + 2, sl]

    pltpu.make_async_copy(ab1, acc.at[isc.at[1]], ss1).wait()

    plsc.subcore_barrier()
    pltpu.sync_copy(acc.at[pl.ds(sid * rows_per, rows_per), :],
                    out.at[pl.ds(cid * acc_pad + sid * rows_per,
                                 rows_per), :])


def _gather_mod_scatter(attr, tab, sct, gat, acc_rows):
    e_pad, h_dim = attr.shape
    acc_pad = _acc_pad_rows(acc_rows)
    nch = e_pad // CHUNK
    mesh = plsc.VectorSubcoreMesh(
        core_axis_name="c", subcore_axis_name="s",
        num_cores=NC, num_subcores=NS)
    kern = pl.kernel(
        functools.partial(_agg_body, acc_pad=acc_pad, nch=nch),
        out_type=jax.ShapeDtypeStruct((NC * acc_pad, h_dim), jnp.float32),
        mesh=mesh,
        scratch_types=[
            pltpu.VMEM_SHARED((acc_pad, h_dim), jnp.float32),
            pltpu.VMEM((4, CHUNK), jnp.int32),
            pltpu.VMEM((4, CHUNK), jnp.int32),
            pltpu.VMEM((2, CHUNK), jnp.int32),
            pltpu.VMEM((CHUNK, h_dim), jnp.float32),
            pltpu.VMEM((CHUNK, h_dim), jnp.float32),
            pltpu.VMEM((CHUNK, h_dim), jnp.float32),
            pltpu.VMEM((CHUNK, h_dim), jnp.float32),
            pltpu.VMEM((ZR, h_dim), jnp.float32),
            pltpu.SemaphoreType.DMA,
            pltpu.SemaphoreType.DMA,
            pltpu.SemaphoreType.DMA,
            pltpu.SemaphoreType.DMA,
            pltpu.SemaphoreType.DMA,
            pltpu.SemaphoreType.DMA,
            pltpu.SemaphoreType.DMA,
        ])
    return kern(attr, tab, sct, gat)


def _addn_body(*refs):
    o_ref = refs[-1]
    acc = refs[0][...]
    for r in refs[1:-1]:
        acc = acc + r[...]
    o_ref[...] = acc


def _add_partials(a, *parts):
    n, h_dim = a.shape
    nb = 1000
    nblk = n // nb
    return pl.pallas_call(
        _addn_body,
        grid=(nblk,),
        in_specs=[pl.BlockSpec((nb, h_dim), lambda i: (i, 0))
                  for _ in range(1 + len(parts))],
        out_specs=pl.BlockSpec((nb, h_dim), lambda i: (i, 0)),
        out_shape=jax.ShapeDtypeStruct((n, h_dim), jnp.float32),
    )(a, *parts)


def _add2_body(p0_ref, p1_ref, o_ref):
    o_ref[...] = p0_ref[...] + p1_ref[...]


def _add_partials_small(p0, p1):
    gp, h_dim = p0.shape
    return pl.pallas_call(
        _add2_body,
        grid=(1,),
        in_specs=[
            pl.BlockSpec((gp, h_dim), lambda i: (0, 0)),
            pl.BlockSpec((gp, h_dim), lambda i: (0, 0)),
        ],
        out_specs=pl.BlockSpec((gp, h_dim), lambda i: (0, 0)),
        out_shape=jax.ShapeDtypeStruct((gp, h_dim), jnp.float32),
    )(p0, p1)


def _head_body(h2_ref, l0_ref, l1_ref, w_ref, b_ref, o_ref):
    h_dim = h2_ref.shape[1]
    lf = l0_ref[...] + l1_ref[...]
    o_ref[...] = (jnp.dot(h2_ref[...], w_ref[0:h_dim, :], precision=HIGH)
                  + jnp.dot(lf, w_ref[h_dim:, :], precision=HIGH)
                  + b_ref[...].reshape(1, -1))


def _head(h2, l0, l1, w_out, b_out):
    n, h_dim = h2.shape
    nb = 1000
    nblk = n // nb
    return pl.pallas_call(
        _head_body,
        grid=(nblk,),
        in_specs=[
            pl.BlockSpec((nb, h_dim), lambda i: (i, 0)),
            pl.BlockSpec((nb, h_dim), lambda i: (i, 0)),
            pl.BlockSpec((nb, h_dim), lambda i: (i, 0)),
            pl.BlockSpec(w_out.shape, lambda i: (0, 0)),
            pl.BlockSpec(b_out.shape, lambda i: (0,)),
        ],
        out_specs=pl.BlockSpec((nb, h_dim), lambda i: (i, 0)),
        out_shape=jax.ShapeDtypeStruct((n, h_dim), jnp.float32),
    )(h2, l0, l1, w_out, b_out)


def kernel(pos, z, edge_index, labels, inter_node, inter_group, emb_table,
           means_s, betas_s, W_rbf_s, b_rbf_s,
           means_l, betas_l, W_rbf_l, b_rbf_l,
           W_out, b_out):
    n = pos.shape[0]
    e = edge_index.shape[1]
    e2 = inter_node.shape[0]
    g = 1000
    gp = _round_up(g, 64)

    nslab = 1
    e_pad = _round_up(e, NW * CHUNK * 2 * nslab)
    e2_pad = _round_up(e2, NW * CHUNK * 2)

    zi = z.astype(jnp.int32)
    z2 = zi.reshape(n, 1)
    pos4 = jnp.concatenate([pos, jnp.ones((n, 1), jnp.float32)], axis=1)
    posx, posy, posz = pos[:, 0], pos[:, 1], pos[:, 2]

    rc0 = jnp.pad(edge_index[0].astype(jnp.int32), (0, e_pad - e))
    rc1 = jnp.pad(edge_index[1].astype(jnp.int32), (0, e_pad - e))
    it0 = jnp.pad(inter_node.astype(jnp.int32), (0, e2_pad - e2))
    it1 = jnp.pad(inter_group.astype(jnp.int32), (0, e2_pad - e2))

    h = _embed(z2, emb_table)
    num = _group_moments(labels, zi, pos4, g)
    numT = jnp.pad(num.T, ((0, 0), (0, gp - g)))

    d2s, d2b = _distances(posx, posy, posz, numT, rc0, rc1, it0, it1,
                          n, gp, e_pad, e2_pad)

    attr_b = _edge_attr(d2b, means_l, betas_l, W_rbf_l, b_rbf_l, CUT_L, e2)

    ap_n = _acc_pad_rows(n)
    ns_ = e_pad // nslab
    parts = []
    for si in range(nslab):
        d2slab = lax.slice(d2s, (si * ns_,), ((si + 1) * ns_,))
        nv = max(0, min(e - si * ns_, ns_))
        attr = _edge_attr(d2slab, means_s, betas_s, W_rbf_s, b_rbf_s,
                          CUT_S, nv)
        hp = _gather_mod_scatter(attr, h,
                                 lax.slice(rc0, (si * ns_,), ((si + 1) * ns_,)),
                                 lax.slice(rc1, (si * ns_,), ((si + 1) * ns_,)),
                                 acc_rows=n)
        parts.extend([hp[:n], hp[ap_n:ap_n + n]])
    h2 = _add_partials(h, *parts)

    ap_g = _acc_pad_rows(gp)
    gparts = _gather_mod_scatter(attr_b, h2, it1, it0, acc_rows=gp)
    g_agg = _add_partials_small(gparts[:ap_g], gparts[ap_g:])

    lparts = _gather_mod_scatter(attr_b, g_agg, it0, it1, acc_rows=n)

    return _head(h2, lparts[:n], lparts[ap_n:ap_n + n], W_out, b_out)


# speedup vs baseline: 1.1632x; 1.0668x over previous
import dataclasses
import functools

import jax
import jax.numpy as jnp
from jax import lax
from jax.experimental import pallas as pl
from jax.experimental.pallas import tpu as pltpu
from jax.experimental.pallas import tpu_sc as plsc

CUT_S = 5.0
CUT_L = 10.0

NC = 2
NS = 16
LN = 16
NW = NC * NS

HIGH = lax.Precision.HIGHEST


def _round_up(x, m):
    return (x + m - 1) // m * m


def _sc_params():
    cp = pltpu.CompilerParams()
    if "needs_layout_passes" in pltpu.CompilerParams.__dataclass_fields__:
        cp = dataclasses.replace(cp, needs_layout_passes=False)
    return cp


def _embed_body(z_ref, emb_ref, h_ref):
    z = z_ref[...]
    n, c = z.shape[0], emb_ref.shape[0]
    iot = lax.broadcasted_iota(jnp.int32, (n, c), 1)
    oh = (z == iot).astype(jnp.float32)
    h_ref[...] = jnp.dot(oh, emb_ref[...], precision=HIGH)


def _embed(z2, emb_table):
    n = z2.shape[0]
    h_dim = emb_table.shape[1]
    return pl.pallas_call(
        _embed_body,
        out_shape=jax.ShapeDtypeStruct((n, h_dim), jnp.float32),
    )(z2, emb_table)


def _gp_body(lab_ref, z_ref, pos_ref, num_ref, *, g, nb):
    i = pl.program_id(0)
    lab = lab_ref[...].reshape(1, nb)
    zf = z_ref[...].reshape(nb, 1).astype(jnp.float32)
    giota = lax.broadcasted_iota(jnp.int32, (g, nb), 0)
    oht = (giota == lab).astype(jnp.float32)
    pw = pos_ref[...].reshape(nb, 4) * zf
    upd = jnp.dot(oht, pw)

    @pl.when(i == 0)
    def _():
        num_ref[...] = upd

    @pl.when(i > 0)
    def _():
        num_ref[...] += upd


def _group_moments(labels, z, pos4, g):
    n = labels.shape[0]
    nb = 1000
    nblk = n // nb
    lab3 = labels.astype(jnp.int32).reshape(nblk, 1, nb)
    z3 = z.astype(jnp.int32).reshape(nblk, 1, nb)
    pos3 = pos4.reshape(nblk, nb, 4)
    return pl.pallas_call(
        functools.partial(_gp_body, g=g, nb=nb),
        grid=(nblk,),
        in_specs=[
            pl.BlockSpec((1, 1, nb), lambda i: (i, 0, 0)),
            pl.BlockSpec((1, 1, nb), lambda i: (i, 0, 0)),
            pl.BlockSpec((1, nb, 4), lambda i: (i, 0, 0)),
        ],
        out_specs=pl.BlockSpec((g, 4), lambda i: (0, 0)),
        out_shape=jax.ShapeDtypeStruct((g, 4), jnp.float32),
    )(lab3, z3, pos3)


def _pick_chunk(total, cap):
    d = min(cap, total) // 8 * 8
    while d >= 8:
        if total % d == 0:
            return d
        d -= 8
    raise ValueError(total)


def _dist_body(posx, posy, posz, numT, rc0, rc1, it0, it1, d2s, d2b,
               px, py, pz, nt, gx, gy, gz, ri, ci, db, ni, gi, db2,
               *, gp, epw, e2pw, ech):
    wid = lax.axis_index("c") * NS + lax.axis_index("s")
    pltpu.sync_copy(posx, px)
    pltpu.sync_copy(posy, py)
    pltpu.sync_copy(posz, pz)
    pltpu.sync_copy(numT, nt)

    @pl.loop(0, gp, step=LN)
    def _(k):
        den = jnp.maximum(nt[3, pl.ds(k, LN)], 1e-9)
        gx[pl.ds(k, LN)] = nt[0, pl.ds(k, LN)] / den
        gy[pl.ds(k, LN)] = nt[1, pl.ds(k, LN)] / den
        gz[pl.ds(k, LN)] = nt[2, pl.ds(k, LN)] / den

    base = wid * epw

    @pl.loop(0, epw, step=ech)
    def _(c0):
        pltpu.sync_copy(rc0.at[pl.ds(base + c0, ech)], ri)
        pltpu.sync_copy(rc1.at[pl.ds(base + c0, ech)], ci)

        @pl.loop(0, ech, step=LN)
        def _(k):
            r16 = ri[pl.ds(k, LN)]
            c16 = ci[pl.ds(k, LN)]
            dx = plsc.load_gather(px, [r16]) - plsc.load_gather(px, [c16])
            dy = plsc.load_gather(py, [r16]) - plsc.load_gather(py, [c16])
            dz = plsc.load_gather(pz, [r16]) - plsc.load_gather(pz, [c16])
            db[pl.ds(k, LN)] = dx * dx + dy * dy + dz * dz

        pltpu.sync_copy(db, d2s.at[pl.ds(base + c0, ech)])

    base2 = wid * e2pw
    pltpu.sync_copy(it0.at[pl.ds(base2, e2pw)], ni)
    pltpu.sync_copy(it1.at[pl.ds(base2, e2pw)], gi)

    @pl.loop(0, e2pw, step=LN)
    def _(k):
        n16 = ni[pl.ds(k, LN)]
        g16 = gi[pl.ds(k, LN)]
        dx = plsc.load_gather(px, [n16]) - plsc.load_gather(gx, [g16])
        dy = plsc.load_gather(py, [n16]) - plsc.load_gather(gy, [g16])
        dz = plsc.load_gather(pz, [n16]) - plsc.load_gather(gz, [g16])
        db2[pl.ds(k, LN)] = dx * dx + dy * dy + dz * dz

    pltpu.sync_copy(db2, d2b.at[pl.ds(base2, e2pw)])


def _distances(posx, posy, posz, numT, rc0, rc1, it0, it1,
               np_, gp, e_pad, e2_pad):
    epw = e_pad // NW
    e2pw = e2_pad // NW
    ech = _pick_chunk(epw, 1440)
    mesh = plsc.VectorSubcoreMesh(
        core_axis_name="c", subcore_axis_name="s",
        num_cores=NC, num_subcores=NS)
    kern = pl.kernel(
        functools.partial(_dist_body, gp=gp, epw=epw, e2pw=e2pw, ech=ech),
        out_type=(jax.ShapeDtypeStruct((e_pad,), jnp.float32),
                  jax.ShapeDtypeStruct((e2_pad,), jnp.float32)),
        mesh=mesh,
        compiler_params=_sc_params(),
        scratch_types=[
            pltpu.VMEM((np_,), jnp.float32),
            pltpu.VMEM((np_,), jnp.float32),
            pltpu.VMEM((np_,), jnp.float32),
            pltpu.VMEM((4, gp), jnp.float32),
            pltpu.VMEM((gp,), jnp.float32),
            pltpu.VMEM((gp,), jnp.float32),
            pltpu.VMEM((gp,), jnp.float32),
            pltpu.VMEM((ech,), jnp.int32),
            pltpu.VMEM((ech,), jnp.int32),
            pltpu.VMEM((ech,), jnp.float32),
            pltpu.VMEM((e2pw,), jnp.int32),
            pltpu.VMEM((e2pw,), jnp.int32),
            pltpu.VMEM((e2pw,), jnp.float32),
        ])
    return kern(posx, posy, posz, numT, rc0, rc1, it0, it1)


def _attr_body(d2_ref, means_ref, betas_ref, w_ref, b_ref, out_ref,
               *, cutoff, n_valid, be):
    i = pl.program_id(0)
    d2 = d2_ref[...].reshape(1, be)
    w = jnp.sqrt(d2 + 1e-12)
    cut = 0.5 * (jnp.cos(w * (jnp.pi / cutoff)) + 1.0)
    cut = jnp.where(w < cutoff, cut, 0.0)
    alpha = 5.0 / cutoff
    log2e = 1.4426950408889634
    u = jnp.exp2((-alpha * log2e) * w)
    means_t = means_ref[...].reshape(-1, 1)
    betas_t = betas_ref[...].reshape(-1, 1)
    t = u - means_t
    rbf_t = cut * jnp.exp2((-log2e) * betas_t * (t * t))
    attr = lax.dot_general(rbf_t, w_ref[...], (((0,), (0,)), ((), ())),
                           precision=HIGH)
    attr = attr + b_ref[...].reshape(1, -1)
    nv_blk = n_valid // be

    @pl.when(i < nv_blk)
    def _():
        out_ref[...] = attr

    @pl.when(i >= nv_blk)
    def _():
        eidx = i * be + lax.broadcasted_iota(jnp.int32, (be, 1), 0)
        out_ref[...] = jnp.where(eidx < n_valid, attr, 0.0)


def _edge_attr(d2, means, betas, w_rbf, b_rbf, cutoff, n_valid):
    e_pad = d2.shape[0]
    h_dim = w_rbf.shape[1]
    be = 2560 if e_pad % 2560 == 0 else (1536 if e_pad % 1536 == 0 else 1024)
    nblk = e_pad // be
    d3 = d2.reshape(nblk, 1, be)
    return pl.pallas_call(
        functools.partial(_attr_body, cutoff=cutoff, n_valid=n_valid, be=be),
        grid=(nblk,),
        in_specs=[
            pl.BlockSpec((1, 1, be), lambda i: (i, 0, 0)),
            pl.BlockSpec(means.shape, lambda i: (0,)),
            pl.BlockSpec(betas.shape, lambda i: (0,)),
            pl.BlockSpec(w_rbf.shape, lambda i: (0, 0)),
            pl.BlockSpec(b_rbf.shape, lambda i: (0,)),
        ],
        out_specs=pl.BlockSpec((be, h_dim), lambda i: (i, 0)),
        out_shape=jax.ShapeDtypeStruct((e_pad, h_dim), jnp.float32),
    )(d3, means, betas, w_rbf, b_rbf)


CHUNK = 80
ZR = 40


def _acc_pad_rows(acc_rows):
    return _round_up(acc_rows, NS * 8)


def _zero_chunks(rows_per):
    offs, o = [], 0
    while o < rows_per:
        sz = min(ZR, rows_per - o)
        offs.append((o, sz))
        o += sz
    return offs


def _agg_body(attr, tab, sct, gat, out, acc,
              isb, igb, isc, ab0, hb0, ab1, hb1, zb,
              si, sa0, sg0, sa1, sg1, ss0, ss1,
              *, acc_pad, nch):
    cid = lax.axis_index("c")
    sid = lax.axis_index("s")
    wid = cid * NS + sid

    @pl.loop(0, ZR)
    def _(i):
        for j in range(8):
            zb[i, pl.ds(j * LN, LN)] = jnp.zeros((LN,), jnp.float32)

    rows_per = acc_pad // NS
    for (zo, zs) in _zero_chunks(rows_per):
        pltpu.sync_copy(zb.at[pl.ds(0, zs), :],
                        acc.at[pl.ds(sid * rows_per + zo, zs), :])

    plsc.subcore_barrier()

    ptot = nch // NS
    pt0 = ptot // 2
    pt1 = ptot - pt0
    per_tile = jnp.where(cid == 0, pt0, pt1)
    npairs = per_tile // 2
    tile_base = jnp.where(cid == 0, sid * pt0,
                          NS * pt0 + sid * pt1) * CHUNK
    data = ((ab0, hb0, sa0, sg0, ss0), (ab1, hb1, sa1, sg1, ss1))

    def idx_copies(p, rbase):
        b0 = tile_base + (2 * p) * CHUNK
        return [(sct.at[pl.ds(b0, CHUNK)], isb.at[rbase]),
                (gat.at[pl.ds(b0, CHUNK)], igb.at[rbase]),
                (sct.at[pl.ds(b0 + CHUNK, CHUNK)], isb.at[rbase + 1]),
                (gat.at[pl.ds(b0 + CHUNK, CHUNK)], igb.at[rbase + 1])]

    def issue_in(k, slot, row):
        ab, hb, sa, sg, _ = data[slot]
        pltpu.async_copy(attr.at[pl.ds(k * CHUNK, CHUNK), :], ab, sa)
        pltpu.async_copy(tab.at[igb.at[row]], hb, sg)

    def mul_and_scatter(slot, row):
        ab, hb, sa, sg, ss = data[slot]
        pltpu.make_async_copy(attr.at[pl.ds(0, CHUNK), :], ab, sa).wait()
        pltpu.make_async_copy(tab.at[igb.at[row]], hb, sg).wait()
        for v in range(CHUNK // LN):
            isc[slot, pl.ds(v * LN, LN)] = isb[row, pl.ds(v * LN, LN)]

        @pl.loop(0, CHUNK)
        def _(i):
            for j in range(8):
                sl = (i, pl.ds(j * LN, LN))
                ab[sl] = ab[sl] * hb[sl]

        pltpu.async_copy(ab, acc.at[isc.at[slot]], ss, add=True)

    for (src, dst) in idx_copies(0, 0):
        pltpu.sync_copy(src, dst)
    issue_in(tile_base // CHUNK + 0, 0, 0)

    @pl.loop(0, npairs)
    def _(j):
        k = tile_base // CHUNK + 2 * j

        @pl.when(j > 0)
        def _():
            pltpu.make_async_copy(ab1, acc.at[isc.at[1]], ss1).wait()

        issue_in(k + 1, 1, 1)
        mul_and_scatter(0, 0)

        @pl.when(j + 1 < npairs)
        def _():
            for (src, dst) in idx_copies(j + 1, 2):
                pltpu.async_copy(src, dst, si)

        pltpu.make_async_copy(ab0, acc.at[isc.at[0]], ss0).wait()

        @pl.when(j + 1 < npairs)
        def _():
            for (src, dst) in idx_copies(j + 1, 2):
                pltpu.make_async_copy(src, dst, si).wait()
            issue_in(k + 2, 0, 2)

        mul_and_scatter(1, 1)

        @pl.when(j + 1 < npairs)
        def _():
            for r in range(2):
                for v in range(CHUNK // LN):
                    sl = pl.ds(v * LN, LN)
                    isb[r, sl] = isb[r + 2, sl]
                    igb[r, sl] = igb[r + 2, sl]

    pltpu.make_async_copy(ab1, acc.at[isc.at[1]], ss1).wait()

    plsc.subcore_barrier()
    pltpu.sync_copy(acc.at[pl.ds(sid * rows_per, rows_per), :],
                    out.at[pl.ds(cid * acc_pad + sid * rows_per,
                                 rows_per), :])


def _gather_mod_scatter(attr, tab, sct, gat, acc_rows):
    e_pad, h_dim = attr.shape
    acc_pad = _acc_pad_rows(acc_rows)
    nch = e_pad // CHUNK
    mesh = plsc.VectorSubcoreMesh(
        core_axis_name="c", subcore_axis_name="s",
        num_cores=NC, num_subcores=NS)
    kern = pl.kernel(
        functools.partial(_agg_body, acc_pad=acc_pad, nch=nch),
        out_type=jax.ShapeDtypeStruct((NC * acc_pad, h_dim), jnp.float32),
        mesh=mesh,
        scratch_types=[
            pltpu.VMEM_SHARED((acc_pad, h_dim), jnp.float32),
            pltpu.VMEM((4, CHUNK), jnp.int32),
            pltpu.VMEM((4, CHUNK), jnp.int32),
            pltpu.VMEM((2, CHUNK), jnp.int32),
            pltpu.VMEM((CHUNK, h_dim), jnp.float32),
            pltpu.VMEM((CHUNK, h_dim), jnp.float32),
            pltpu.VMEM((CHUNK, h_dim), jnp.float32),
            pltpu.VMEM((CHUNK, h_dim), jnp.float32),
            pltpu.VMEM((ZR, h_dim), jnp.float32),
            pltpu.SemaphoreType.DMA,
            pltpu.SemaphoreType.DMA,
            pltpu.SemaphoreType.DMA,
            pltpu.SemaphoreType.DMA,
            pltpu.SemaphoreType.DMA,
            pltpu.SemaphoreType.DMA,
            pltpu.SemaphoreType.DMA,
        ])
    return kern(attr, tab, sct, gat)


def _addn_body(*refs):
    o_ref = refs[-1]
    acc = refs[0][...]
    for r in refs[1:-1]:
        acc = acc + r[...]
    o_ref[...] = acc


def _add_partials(a, *parts):
    n, h_dim = a.shape
    nb = 1000
    nblk = n // nb
    return pl.pallas_call(
        _addn_body,
        grid=(nblk,),
        in_specs=[pl.BlockSpec((nb, h_dim), lambda i: (i, 0))
                  for _ in range(1 + len(parts))],
        out_specs=pl.BlockSpec((nb, h_dim), lambda i: (i, 0)),
        out_shape=jax.ShapeDtypeStruct((n, h_dim), jnp.float32),
    )(a, *parts)


def _add2_body(p0_ref, p1_ref, o_ref):
    o_ref[...] = p0_ref[...] + p1_ref[...]


def _add_partials_small(p0, p1):
    gp, h_dim = p0.shape
    return pl.pallas_call(
        _add2_body,
        grid=(1,),
        in_specs=[
            pl.BlockSpec((gp, h_dim), lambda i: (0, 0)),
            pl.BlockSpec((gp, h_dim), lambda i: (0, 0)),
        ],
        out_specs=pl.BlockSpec((gp, h_dim), lambda i: (0, 0)),
        out_shape=jax.ShapeDtypeStruct((gp, h_dim), jnp.float32),
    )(p0, p1)


def _head_body(h2_ref, l0_ref, l1_ref, w_ref, b_ref, o_ref):
    h_dim = h2_ref.shape[1]
    lf = l0_ref[...] + l1_ref[...]
    o_ref[...] = (jnp.dot(h2_ref[...], w_ref[0:h_dim, :], precision=HIGH)
                  + jnp.dot(lf, w_ref[h_dim:, :], precision=HIGH)
                  + b_ref[...].reshape(1, -1))


def _head(h2, l0, l1, w_out, b_out):
    n, h_dim = h2.shape
    nb = 1000
    nblk = n // nb
    return pl.pallas_call(
        _head_body,
        grid=(nblk,),
        in_specs=[
            pl.BlockSpec((nb, h_dim), lambda i: (i, 0)),
            pl.BlockSpec((nb, h_dim), lambda i: (i, 0)),
            pl.BlockSpec((nb, h_dim), lambda i: (i, 0)),
            pl.BlockSpec(w_out.shape, lambda i: (0, 0)),
            pl.BlockSpec(b_out.shape, lambda i: (0,)),
        ],
        out_specs=pl.BlockSpec((nb, h_dim), lambda i: (i, 0)),
        out_shape=jax.ShapeDtypeStruct((n, h_dim), jnp.float32),
    )(h2, l0, l1, w_out, b_out)


def kernel(pos, z, edge_index, labels, inter_node, inter_group, emb_table,
           means_s, betas_s, W_rbf_s, b_rbf_s,
           means_l, betas_l, W_rbf_l, b_rbf_l,
           W_out, b_out):
    n = pos.shape[0]
    e = edge_index.shape[1]
    e2 = inter_node.shape[0]
    g = 1000
    gp = _round_up(g, 64)

    nslab = 1
    e_pad = _round_up(e, NW * CHUNK * 2 * nslab)
    e2_pad = _round_up(e2, NW * CHUNK * 2)

    zi = z.astype(jnp.int32)
    z2 = zi.reshape(n, 1)
    pos4 = jnp.concatenate([pos, jnp.ones((n, 1), jnp.float32)], axis=1)
    posx, posy, posz = pos[:, 0], pos[:, 1], pos[:, 2]

    rc0 = jnp.pad(edge_index[0].astype(jnp.int32), (0, e_pad - e))
    rc1 = jnp.pad(edge_index[1].astype(jnp.int32), (0, e_pad - e))
    it0 = jnp.pad(inter_node.astype(jnp.int32), (0, e2_pad - e2))
    it1 = jnp.pad(inter_group.astype(jnp.int32), (0, e2_pad - e2))

    h = _embed(z2, emb_table)
    num = _group_moments(labels, zi, pos4, g)
    numT = jnp.pad(num.T, ((0, 0), (0, gp - g)))

    d2s, d2b = _distances(posx, posy, posz, numT, rc0, rc1, it0, it1,
                          n, gp, e_pad, e2_pad)

    attr_b = _edge_attr(d2b, means_l, betas_l, W_rbf_l, b_rbf_l, CUT_L, e2)

    ap_n = _acc_pad_rows(n)
    ns_ = e_pad // nslab
    parts = []
    for si in range(nslab):
        d2slab = lax.slice(d2s, (si * ns_,), ((si + 1) * ns_,))
        nv = max(0, min(e - si * ns_, ns_))
        attr = _edge_attr(d2slab, means_s, betas_s, W_rbf_s, b_rbf_s,
                          CUT_S, nv)
        hp = _gather_mod_scatter(attr, h,
                                 lax.slice(rc0, (si * ns_,), ((si + 1) * ns_,)),
                                 lax.slice(rc1, (si * ns_,), ((si + 1) * ns_,)),
                                 acc_rows=n)
        parts.extend([hp[:n], hp[ap_n:ap_n + n]])
    h2 = _add_partials(h, *parts)

    ap_g = _acc_pad_rows(gp)
    gparts = _gather_mod_scatter(attr_b, h2, it1, it0, acc_rows=gp)
    g_agg = _add_partials_small(gparts[:ap_g], gparts[ap_g:])

    lparts = _gather_mod_scatter(attr_b, g_agg, it0, it1, acc_rows=n)

    return _head(h2, lparts[:n], lparts[ap_n:ap_n + n], W_out, b_out)
